# Initial kernel scaffold; baseline (speedup 1.0000x reference)
import jax
import jax.numpy as jnp
from jax.experimental import pallas as pl


def kernel(x, edge_index, W1, b1, W2, b2):
    raise NotImplementedError("write your pallas kernel here")


# speedup vs baseline: 30.7079x; 30.7079x over previous
import jax
import jax.numpy as jnp
from jax import lax
from jax.experimental import pallas as pl
from jax.experimental.pallas import tpu as pltpu
from jax.experimental.pallas import tpu_sc as plsc

_N = 10000
_E = 320000
_D = 128
_DH = _D // 2

_NC = 2
_NS = 16
_NW = _NC * _NS

_CHUNK = 80
_NBUF = 5

_EPW_DEG = _E // _NW
_NCH_DEG = _EPW_DEG // _CHUNK

_EPS = _E // _NS
_NCH = _EPS // _CHUNK

_NPAD = 10240
_RPT = _NPAD // _NS

_BM = 2048
_GRID = _NPAD // _BM

_MESH = plsc.VectorSubcoreMesh(
    core_axis_name="c", subcore_axis_name="s", num_cores=_NC, num_subcores=_NS
)
_SC_PARAMS = pltpu.CompilerParams(use_tc_tiling_on_sc=False)


def _sc_degree_body(dst_hbm, ones_hbm, z16_hbm, deg_hbm, dst_v, ones_v, deg_sh):
    c = lax.axis_index("c")
    s = lax.axis_index("s")
    wid = s * _NC + c
    pltpu.sync_copy(dst_hbm.at[wid], dst_v)
    pltpu.sync_copy(ones_hbm, ones_v)
    base = s * _RPT
    for k in range(_RPT // _CHUNK):
        pltpu.sync_copy(z16_hbm, deg_sh.at[pl.ds(base + k * _CHUNK, _CHUNK)])
    plsc.subcore_barrier()

    def chunk(j, carry):
        pltpu.sync_copy(ones_v, deg_sh.at[dst_v.at[j]], add=True)
        return carry

    lax.fori_loop(0, _NCH_DEG, chunk, 0)
    plsc.subcore_barrier()
    out_base = c * _NPAD + base
    for k in range(_RPT // _CHUNK):
        pltpu.sync_copy(
            deg_sh.at[pl.ds(base + k * _CHUNK, _CHUNK)],
            deg_hbm.at[pl.ds(out_base + k * _CHUNK, _CHUNK)],
        )


_sc_degree = pl.kernel(
    _sc_degree_body,
    out_type=jax.ShapeDtypeStruct((_NC * _NPAD, 16), jnp.float32),
    mesh=_MESH,
    scratch_types=[
        pltpu.VMEM((_NCH_DEG, _CHUNK), jnp.int32),
        pltpu.VMEM((_CHUNK, 16), jnp.float32),
        pltpu.VMEM_SHARED((_NPAD, 16), jnp.float32),
    ],
    compiler_params=_SC_PARAMS,
)


def _sc_scatter_body(
    g2n_hbm, src2_hbm, dst_hbm, z64_hbm, acc_hbm,
    src_v, dst_v, r0, r1, r2, r3, r4, acc_sh, s0, s1, s2, s3, s4
):
    c = lax.axis_index("c")
    s = lax.axis_index("s")
    rbufs = (r0, r1, r2, r3, r4)
    sems = (s0, s1, s2, s3, s4)
    pltpu.sync_copy(src2_hbm.at[c * _NS + s], src_v)
    pltpu.sync_copy(dst_hbm.at[s], dst_v)
    base = s * _RPT
    for k in range(_RPT // _CHUNK):
        pltpu.sync_copy(z64_hbm, acc_sh.at[pl.ds(base + k * _CHUNK, _CHUNK)])
    plsc.subcore_barrier()

    for b in range(_NBUF):
        pltpu.async_copy(g2n_hbm.at[src_v.at[b]], rbufs[b], sems[b])

    def outer(o, carry):
        for b in range(_NBUF):
            cc = o * _NBUF + b
            pltpu.make_async_copy(g2n_hbm.at[src_v.at[cc]], rbufs[b], sems[b]).wait()
            pltpu.sync_copy(rbufs[b], acc_sh.at[dst_v.at[cc]], add=True)
            nxt = cc + _NBUF

            @pl.when(nxt < _NCH)
            def _start_next():
                pltpu.async_copy(g2n_hbm.at[src_v.at[nxt]], rbufs[b], sems[b])

        return carry

    lax.fori_loop(0, _NCH // _NBUF, outer, 0)
    plsc.subcore_barrier()
    out_base = c * _NPAD + base
    for k in range(_RPT // _CHUNK):
        pltpu.sync_copy(
            acc_sh.at[pl.ds(base + k * _CHUNK, _CHUNK)],
            acc_hbm.at[pl.ds(out_base + k * _CHUNK, _CHUNK)],
        )


_sc_scatter = pl.kernel(
    _sc_scatter_body,
    out_type=jax.ShapeDtypeStruct((_NC * _NPAD, _DH), jnp.float32),
    mesh=_MESH,
    scratch_types=[
        pltpu.VMEM((_NCH, _CHUNK), jnp.int32),
        pltpu.VMEM((_NCH, _CHUNK), jnp.int32),
        pltpu.VMEM((_CHUNK, _DH), jnp.float32),
        pltpu.VMEM((_CHUNK, _DH), jnp.float32),
        pltpu.VMEM((_CHUNK, _DH), jnp.float32),
        pltpu.VMEM((_CHUNK, _DH), jnp.float32),
        pltpu.VMEM((_CHUNK, _DH), jnp.float32),
        pltpu.VMEM_SHARED((_NPAD, _DH), jnp.float32),
        pltpu.SemaphoreType.DMA,
        pltpu.SemaphoreType.DMA,
        pltpu.SemaphoreType.DMA,
        pltpu.SemaphoreType.DMA,
        pltpu.SemaphoreType.DMA,
    ],
    compiler_params=_SC_PARAMS,
)


def _tc_a_body(deg_ref, x_ref, w_ref, g_ref, dinv_ref):
    dp = deg_ref[...]
    deg = dp[0, :, 0:1] + dp[1, :, 0:1] + 1.0
    dinv = lax.rsqrt(deg)
    h = jnp.dot(x_ref[...], w_ref[...], precision=lax.Precision.HIGHEST,
                preferred_element_type=jnp.float32)
    g_ref[...] = h * dinv
    dinv_ref[...] = dinv


def _tc_b_body(acc_ref, g_ref, dinv_ref, b_ref, w_ref, g2_ref):
    dinv = dinv_ref[...]
    acc = jnp.concatenate([acc_ref[0], acc_ref[1]], axis=1)
    z = jnp.maximum(dinv * (acc + g_ref[...]) + b_ref[...], 0.0)
    h2 = jnp.dot(z, w_ref[...], precision=lax.Precision.HIGHEST,
                 preferred_element_type=jnp.float32)
    g2_ref[...] = h2 * dinv


def _tc_c_body(acc_ref, g_ref, dinv_ref, b_ref, o_ref):
    acc = jnp.concatenate([acc_ref[0], acc_ref[1]], axis=1)
    o_ref[...] = dinv_ref[...] * (acc + g_ref[...]) + b_ref[...]


def _tc_a(deg_parts, x, W1):
    return pl.pallas_call(
        _tc_a_body,
        grid=(_GRID,),
        in_specs=[
            pl.BlockSpec((_NC, _BM, 16), lambda i: (0, i, 0)),
            pl.BlockSpec((_BM, _D), lambda i: (i, 0)),
            pl.BlockSpec((_D, _D), lambda i: (0, 0)),
        ],
        out_specs=[
            pl.BlockSpec((_BM, _D), lambda i: (i, 0)),
            pl.BlockSpec((_BM, 1), lambda i: (i, 0)),
        ],
        out_shape=[
            jax.ShapeDtypeStruct((_N, _D), jnp.float32),
            jax.ShapeDtypeStruct((_N, 1), jnp.float32),
        ],
    )(deg_parts, x, W1)


def _tc_b(acc_parts, g1, dinv, b1, W2):
    return pl.pallas_call(
        _tc_b_body,
        grid=(_GRID,),
        in_specs=[
            pl.BlockSpec((_NC, _BM, _DH), lambda i: (0, i, 0)),
            pl.BlockSpec((_BM, _D), lambda i: (i, 0)),
            pl.BlockSpec((_BM, 1), lambda i: (i, 0)),
            pl.BlockSpec((1, _D), lambda i: (0, 0)),
            pl.BlockSpec((_D, _D), lambda i: (0, 0)),
        ],
        out_specs=pl.BlockSpec((_BM, _D), lambda i: (i, 0)),
        out_shape=jax.ShapeDtypeStruct((_N, _D), jnp.float32),
    )(acc_parts, g1, dinv, b1, W2)


def _tc_c(acc_parts, g2, dinv, b2):
    return pl.pallas_call(
        _tc_c_body,
        grid=(_GRID,),
        in_specs=[
            pl.BlockSpec((_NC, _BM, _DH), lambda i: (0, i, 0)),
            pl.BlockSpec((_BM, _D), lambda i: (i, 0)),
            pl.BlockSpec((_BM, 1), lambda i: (i, 0)),
            pl.BlockSpec((1, _D), lambda i: (0, 0)),
        ],
        out_specs=pl.BlockSpec((_BM, _D), lambda i: (i, 0)),
        out_shape=jax.ShapeDtypeStruct((_N, _D), jnp.float32),
    )(acc_parts, g2, dinv, b2)


def kernel(x, edge_index, W1, b1, W2, b2):
    ei = edge_index.astype(jnp.int32)
    dst_deg = ei[1].reshape(_NW, _NCH_DEG, _CHUNK)
    src_s = ei[0].reshape(_NS, _NCH, _CHUNK)
    dst_s = ei[1].reshape(_NS, _NCH, _CHUNK)
    src2 = jnp.stack([2 * src_s, 2 * src_s + 1]).reshape(_NC * _NS, _NCH, _CHUNK)
    ones16 = jnp.ones((_CHUNK, 16), jnp.float32)
    z16 = jnp.zeros((_CHUNK, 16), jnp.float32)
    z64 = jnp.zeros((_CHUNK, _DH), jnp.float32)
    b1r = b1.reshape(1, _D)
    b2r = b2.reshape(1, _D)

    deg_parts = _sc_degree(dst_deg, ones16, z16).reshape(_NC, _NPAD, 16)
    g1, dinv = _tc_a(deg_parts, x, W1)
    acc1 = _sc_scatter(g1.reshape(2 * _N, _DH), src2, dst_s, z64)
    g2 = _tc_b(acc1.reshape(_NC, _NPAD, _DH), g1, dinv, b1r, W2)
    acc2 = _sc_scatter(g2.reshape(2 * _N, _DH), src2, dst_s, z64)
    return _tc_c(acc2.reshape(_NC, _NPAD, _DH), g2, dinv, b2r)
